# Initial kernel scaffold; baseline (speedup 1.0000x reference)
import jax
import jax.numpy as jnp
from jax.experimental import pallas as pl


def kernel(x, edge_index, edge_values, filters):
    raise NotImplementedError("write your pallas kernel here")


# speedup vs baseline: 6.4348x; 6.4348x over previous
import functools

import jax
import jax.numpy as jnp
from jax import lax
from jax.experimental import pallas as pl
from jax.experimental.pallas import tpu as pltpu
from jax.experimental.pallas import tpu_sc as plsc

N = 10000
D = 128
F = 128
E = 320000

NC = 2
NS = 16
NW = NC * NS
L = 16

EW = E // NW
C = 80
NCHUNK = EW // C
GC = 25
NG = NCHUNK // GC

RPS = 624
TAIL = N - NS * RPS
ZR = RPS // 6

MM_BLK = 1000


def _mm_body(x_ref, f_ref, o_ref):
    o_ref[...] = jnp.dot(x_ref[...], f_ref[...],
                         preferred_element_type=jnp.float32)


def _matmul(x, filters):
    return pl.pallas_call(
        _mm_body,
        grid=(N // MM_BLK,),
        in_specs=[
            pl.BlockSpec((MM_BLK, D), lambda i: (i, 0)),
            pl.BlockSpec((D, F), lambda i: (0, 0)),
        ],
        out_specs=pl.BlockSpec((MM_BLK, F), lambda i: (i, 0)),
        out_shape=jax.ShapeDtypeStruct((N, F), jnp.float32),
    )(x, filters)


def _add_body(p_ref, o_ref):
    o_ref[...] = p_ref[0] + p_ref[1]


def _combine(partials):
    return pl.pallas_call(
        _add_body,
        grid=(N // MM_BLK,),
        in_specs=[pl.BlockSpec((NC, MM_BLK, F), lambda i: (0, i, 0))],
        out_specs=pl.BlockSpec((MM_BLK, F), lambda i: (i, 0)),
        out_shape=jax.ShapeDtypeStruct((N, F), jnp.float32),
    )(partials)


def _sc_body(xf_hbm, cols_hbm, rows_hbm, ev_hbm, out_hbm,
             cols_v, rows_v, ev_v, buf0, buf1, zbuf, acc, sem0, sem1):
    c = lax.axis_index("c")
    s = lax.axis_index("s")
    wid = c * NS + s

    zeros16 = jnp.zeros((L,), jnp.float32)

    def zero_row(i, _):
        for j in range(F // L):
            zbuf[i, pl.ds(j * L, L)] = zeros16
        return 0

    lax.fori_loop(0, ZR, zero_row, 0)
    for k in range(RPS // ZR):
        pltpu.sync_copy(zbuf, acc.at[pl.ds(s * RPS + k * ZR, ZR)])

    @pl.when(s == NS - 1)
    def _zero_tail():
        pltpu.sync_copy(zbuf.at[pl.ds(0, TAIL)], acc.at[pl.ds(NS * RPS, TAIL)])

    plsc.subcore_barrier()

    def scale(buf, i):
        for g in range(C // L):
            ev16 = ev_v[i, pl.ds(g * L, L)]
            for k in range(L):
                gb = jnp.broadcast_to(ev16[k], (L,))
                e = g * L + k
                for j in range(F // L):
                    buf[e, pl.ds(j * L, L)] = buf[e, pl.ds(j * L, L)] * gb

    def process(i, _):
        pltpu.async_copy(xf_hbm.at[cols_v.at[i]], buf0, sem0).wait()
        scale(buf0, i)
        pltpu.sync_copy(buf0, acc.at[rows_v.at[i]], add=True)
        return 0

    def group(g, _):
        pltpu.sync_copy(cols_hbm.at[wid, g], cols_v)
        pltpu.sync_copy(rows_hbm.at[wid, g], rows_v)
        pltpu.sync_copy(ev_hbm.at[wid, g], ev_v)
        lax.fori_loop(0, GC, process, 0)
        return 0

    lax.fori_loop(0, NG, group, 0)

    plsc.subcore_barrier()
    plsc.subcore_barrier()
    pltpu.sync_copy(acc.at[pl.ds(s * RPS, RPS)],
                    out_hbm.at[c, pl.ds(s * RPS, RPS)])

    @pl.when(s == NS - 1)
    def _write_tail():
        pltpu.sync_copy(acc.at[pl.ds(NS * RPS, TAIL)],
                        out_hbm.at[c, pl.ds(NS * RPS, TAIL)])


_sc_kernel = functools.partial(
    pl.kernel,
    out_type=jax.ShapeDtypeStruct((NC, N, F), jnp.float32),
    mesh=plsc.VectorSubcoreMesh(core_axis_name="c", subcore_axis_name="s",
                                num_cores=NC, num_subcores=NS),
    scratch_types=[
        pltpu.VMEM((GC, C), jnp.int32),
        pltpu.VMEM((GC, C), jnp.int32),
        pltpu.VMEM((GC, C), jnp.float32),
        pltpu.VMEM((C, F), jnp.float32),
        pltpu.VMEM((C, F), jnp.float32),
        pltpu.VMEM((ZR, F), jnp.float32),
        pltpu.VMEM_SHARED((N, F), jnp.float32),
        pltpu.SemaphoreType.DMA,
        pltpu.SemaphoreType.DMA,
    ],
)(_sc_body)


@jax.jit
def kernel(x, edge_index, edge_values, filters):
    xf = _matmul(x, filters)
    rows4 = edge_index[0].reshape(NW, NG, GC, C)
    cols4 = edge_index[1].reshape(NW, NG, GC, C)
    ev4 = edge_values.reshape(NW, NG, GC, C)
    partials = _sc_kernel(xf, cols4, rows4, ev4)
    return _combine(partials)


# speedup vs baseline: 8.3696x; 1.3007x over previous
import functools

import jax
import jax.numpy as jnp
from jax import lax
from jax.experimental import pallas as pl
from jax.experimental.pallas import tpu as pltpu
from jax.experimental.pallas import tpu_sc as plsc

N = 10000
D = 128
F = 128
E = 320000

NC = 2
NS = 16
NW = NC * NS
L = 16

EW = E // NW
C = 80
NCHUNK = EW // C
GC = 25
NG = NCHUNK // GC

RPS = 624
TAIL = N - NS * RPS
ZR = RPS // 6

MM_BLK = 1000


def _mm_body(x_ref, f_ref, o_ref):
    o_ref[...] = jnp.dot(x_ref[...], f_ref[...],
                         preferred_element_type=jnp.float32)


def _matmul(x, filters):
    return pl.pallas_call(
        _mm_body,
        grid=(N // MM_BLK,),
        in_specs=[
            pl.BlockSpec((MM_BLK, D), lambda i: (i, 0)),
            pl.BlockSpec((D, F), lambda i: (0, 0)),
        ],
        out_specs=pl.BlockSpec((MM_BLK, F), lambda i: (i, 0)),
        out_shape=jax.ShapeDtypeStruct((N, F), jnp.float32),
    )(x, filters)


def _add_body(p_ref, o_ref):
    o_ref[...] = p_ref[0] + p_ref[1]


def _combine(partials):
    return pl.pallas_call(
        _add_body,
        grid=(N // MM_BLK,),
        in_specs=[pl.BlockSpec((NC, MM_BLK, F), lambda i: (0, i, 0))],
        out_specs=pl.BlockSpec((MM_BLK, F), lambda i: (i, 0)),
        out_shape=jax.ShapeDtypeStruct((N, F), jnp.float32),
    )(partials)


def _sc_body(xf_hbm, cols_hbm, rows_hbm, ev_hbm, out_hbm,
             cols_v, rows_v, ev_v, buf0, buf1, zbuf, acc, sem0, sem1):
    c = lax.axis_index("c")
    s = lax.axis_index("s")
    wid = c * NS + s

    zeros16 = jnp.zeros((L,), jnp.float32)

    def zero_row(i, _):
        for j in range(F // L):
            zbuf[i, pl.ds(j * L, L)] = zeros16
        return 0

    lax.fori_loop(0, ZR, zero_row, 0)
    for k in range(RPS // ZR):
        pltpu.sync_copy(zbuf, acc.at[pl.ds(s * RPS + k * ZR, ZR)])

    @pl.when(s == NS - 1)
    def _zero_tail():
        pltpu.sync_copy(zbuf.at[pl.ds(0, TAIL)], acc.at[pl.ds(NS * RPS, TAIL)])

    plsc.subcore_barrier()

    def scale(buf, i):
        for g in range(C // L):
            ev16 = ev_v[i, pl.ds(g * L, L)]
            for k in range(L):
                gb = jnp.broadcast_to(ev16[k], (L,))
                e = g * L + k
                for j in range(F // L):
                    buf[e, pl.ds(j * L, L)] = buf[e, pl.ds(j * L, L)] * gb

    def start_gather(i, buf, sem):
        pltpu.async_copy(xf_hbm.at[cols_v.at[i]], buf, sem)

    def wait_gather(i, buf, sem):
        pltpu.make_async_copy(xf_hbm.at[cols_v.at[i]], buf, sem).wait()

    def finish(i, buf):
        scale(buf, i)
        pltpu.sync_copy(buf, acc.at[rows_v.at[i]], add=True)

    def group(g, _):
        pltpu.sync_copy(cols_hbm.at[wid, g], cols_v)
        pltpu.sync_copy(rows_hbm.at[wid, g], rows_v)
        pltpu.sync_copy(ev_hbm.at[wid, g], ev_v)

        start_gather(0, buf0, sem0)

        def pair(j, _):
            i0 = 2 * j
            wait_gather(i0, buf0, sem0)
            start_gather(i0 + 1, buf1, sem1)
            finish(i0, buf0)
            wait_gather(i0 + 1, buf1, sem1)
            start_gather(i0 + 2, buf0, sem0)
            finish(i0 + 1, buf1)
            return 0

        lax.fori_loop(0, (GC - 1) // 2, pair, 0)
        wait_gather(GC - 1, buf0, sem0)
        finish(GC - 1, buf0)
        return 0

    lax.fori_loop(0, NG, group, 0)

    plsc.subcore_barrier()
    plsc.subcore_barrier()
    pltpu.sync_copy(acc.at[pl.ds(s * RPS, RPS)],
                    out_hbm.at[c, pl.ds(s * RPS, RPS)])

    @pl.when(s == NS - 1)
    def _write_tail():
        pltpu.sync_copy(acc.at[pl.ds(NS * RPS, TAIL)],
                        out_hbm.at[c, pl.ds(NS * RPS, TAIL)])


_sc_kernel = functools.partial(
    pl.kernel,
    out_type=jax.ShapeDtypeStruct((NC, N, F), jnp.float32),
    mesh=plsc.VectorSubcoreMesh(core_axis_name="c", subcore_axis_name="s",
                                num_cores=NC, num_subcores=NS),
    scratch_types=[
        pltpu.VMEM((GC, C), jnp.int32),
        pltpu.VMEM((GC, C), jnp.int32),
        pltpu.VMEM((GC, C), jnp.float32),
        pltpu.VMEM((C, F), jnp.float32),
        pltpu.VMEM((C, F), jnp.float32),
        pltpu.VMEM((ZR, F), jnp.float32),
        pltpu.VMEM_SHARED((N, F), jnp.float32),
        pltpu.SemaphoreType.DMA,
        pltpu.SemaphoreType.DMA,
    ],
)(_sc_body)


@jax.jit
def kernel(x, edge_index, edge_values, filters):
    xf = _matmul(x, filters)
    rows4 = edge_index[0].reshape(NW, NG, GC, C)
    cols4 = edge_index[1].reshape(NW, NG, GC, C)
    ev4 = edge_values.reshape(NW, NG, GC, C)
    partials = _sc_kernel(xf, cols4, rows4, ev4)
    return _combine(partials)


# speedup vs baseline: 8.4525x; 1.0099x over previous
import functools

import jax
import jax.numpy as jnp
from jax import lax
from jax.experimental import pallas as pl
from jax.experimental.pallas import tpu as pltpu
from jax.experimental.pallas import tpu_sc as plsc

N = 10000
D = 128
F = 128
E = 320000

NC = 2
NS = 16
NW = NC * NS
L = 16

EW = E // NW
C = 80
NCHUNK = EW // C
GC = 25
NG = NCHUNK // GC

RPS = 624
TAIL = N - NS * RPS
ZR = RPS // 6

MM_BLK = 1000


def _mm_body(x_ref, f_ref, o_ref):
    o_ref[...] = jnp.dot(x_ref[...], f_ref[...],
                         preferred_element_type=jnp.float32)


def _matmul(x, filters):
    return pl.pallas_call(
        _mm_body,
        grid=(N // MM_BLK,),
        in_specs=[
            pl.BlockSpec((MM_BLK, D), lambda i: (i, 0)),
            pl.BlockSpec((D, F), lambda i: (0, 0)),
        ],
        out_specs=pl.BlockSpec((MM_BLK, F), lambda i: (i, 0)),
        out_shape=jax.ShapeDtypeStruct((N, F), jnp.float32),
    )(x, filters)


def _add_body(p_ref, o_ref):
    o_ref[...] = p_ref[0] + p_ref[1]


def _combine(partials):
    return pl.pallas_call(
        _add_body,
        grid=(N // MM_BLK,),
        in_specs=[pl.BlockSpec((NC, MM_BLK, F), lambda i: (0, i, 0))],
        out_specs=pl.BlockSpec((MM_BLK, F), lambda i: (i, 0)),
        out_shape=jax.ShapeDtypeStruct((N, F), jnp.float32),
    )(partials)


def _sc_body(xf_hbm, cols_hbm, rows_hbm, ev_hbm, out_hbm,
             cols_v, rows_v, ev_v, buf0, buf1, zbuf, acc,
             sem0, sem1, ssem0, ssem1):
    c = lax.axis_index("c")
    s = lax.axis_index("s")
    wid = c * NS + s

    zeros16 = jnp.zeros((L,), jnp.float32)

    def zero_row(i, _):
        for j in range(F // L):
            zbuf[i, pl.ds(j * L, L)] = zeros16
        return 0

    lax.fori_loop(0, ZR, zero_row, 0)
    for k in range(RPS // ZR):
        pltpu.sync_copy(zbuf, acc.at[pl.ds(s * RPS + k * ZR, ZR)])

    @pl.when(s == NS - 1)
    def _zero_tail():
        pltpu.sync_copy(zbuf.at[pl.ds(0, TAIL)], acc.at[pl.ds(NS * RPS, TAIL)])

    plsc.subcore_barrier()

    def scale(buf, i):
        for g in range(C // L):
            ev16 = ev_v[i, pl.ds(g * L, L)]
            for k in range(L):
                gb = jnp.broadcast_to(ev16[k], (L,))
                e = g * L + k
                for j in range(F // L):
                    buf[e, pl.ds(j * L, L)] = buf[e, pl.ds(j * L, L)] * gb

    def start_gather(i, buf, sem):
        pltpu.async_copy(xf_hbm.at[cols_v.at[i]], buf, sem)

    def wait_gather(i, buf, sem):
        pltpu.make_async_copy(xf_hbm.at[cols_v.at[i]], buf, sem).wait()

    def start_scatter(i, buf, sem):
        pltpu.async_copy(buf, acc.at[rows_v.at[i]], sem, add=True)

    def wait_scatter(i, buf, sem):
        pltpu.make_async_copy(buf, acc.at[rows_v.at[i]], sem).wait()

    def group(g, _):
        pltpu.sync_copy(cols_hbm.at[wid, g], cols_v)
        pltpu.sync_copy(rows_hbm.at[wid, g], rows_v)
        pltpu.sync_copy(ev_hbm.at[wid, g], ev_v)

        start_gather(0, buf0, sem0)
        wait_gather(0, buf0, sem0)
        start_gather(1, buf1, sem1)
        scale(buf0, 0)
        start_scatter(0, buf0, ssem0)

        def pair(j, _):
            i1 = 2 * j + 1
            wait_gather(i1, buf1, sem1)
            wait_scatter(i1 - 1, buf0, ssem0)
            start_gather(i1 + 1, buf0, sem0)
            scale(buf1, i1)
            start_scatter(i1, buf1, ssem1)
            wait_gather(i1 + 1, buf0, sem0)
            wait_scatter(i1, buf1, ssem1)
            start_gather(i1 + 2, buf1, sem1)
            scale(buf0, i1 + 1)
            start_scatter(i1 + 1, buf0, ssem0)
            return 0

        lax.fori_loop(0, (GC - 3) // 2, pair, 0)
        wait_gather(GC - 2, buf1, sem1)
        wait_scatter(GC - 3, buf0, ssem0)
        start_gather(GC - 1, buf0, sem0)
        scale(buf1, GC - 2)
        start_scatter(GC - 2, buf1, ssem1)
        wait_gather(GC - 1, buf0, sem0)
        scale(buf0, GC - 1)
        start_scatter(GC - 1, buf0, ssem0)
        wait_scatter(GC - 2, buf1, ssem1)
        wait_scatter(GC - 1, buf0, ssem0)
        return 0

    lax.fori_loop(0, NG, group, 0)

    plsc.subcore_barrier()
    plsc.subcore_barrier()
    pltpu.sync_copy(acc.at[pl.ds(s * RPS, RPS)],
                    out_hbm.at[c, pl.ds(s * RPS, RPS)])

    @pl.when(s == NS - 1)
    def _write_tail():
        pltpu.sync_copy(acc.at[pl.ds(NS * RPS, TAIL)],
                        out_hbm.at[c, pl.ds(NS * RPS, TAIL)])


_sc_kernel = functools.partial(
    pl.kernel,
    out_type=jax.ShapeDtypeStruct((NC, N, F), jnp.float32),
    mesh=plsc.VectorSubcoreMesh(core_axis_name="c", subcore_axis_name="s",
                                num_cores=NC, num_subcores=NS),
    scratch_types=[
        pltpu.VMEM((GC, C), jnp.int32),
        pltpu.VMEM((GC, C), jnp.int32),
        pltpu.VMEM((GC, C), jnp.float32),
        pltpu.VMEM((C, F), jnp.float32),
        pltpu.VMEM((C, F), jnp.float32),
        pltpu.VMEM((ZR, F), jnp.float32),
        pltpu.VMEM_SHARED((N, F), jnp.float32),
        pltpu.SemaphoreType.DMA,
        pltpu.SemaphoreType.DMA,
        pltpu.SemaphoreType.DMA,
        pltpu.SemaphoreType.DMA,
    ],
)(_sc_body)


@jax.jit
def kernel(x, edge_index, edge_values, filters):
    xf = _matmul(x, filters)
    rows4 = edge_index[0].reshape(NW, NG, GC, C)
    cols4 = edge_index[1].reshape(NW, NG, GC, C)
    ev4 = edge_values.reshape(NW, NG, GC, C)
    partials = _sc_kernel(xf, cols4, rows4, ev4)
    return _combine(partials)


# speedup vs baseline: 10.7248x; 1.2688x over previous
import functools

import jax
import jax.numpy as jnp
from jax import lax
from jax.experimental import pallas as pl
from jax.experimental.pallas import tpu as pltpu
from jax.experimental.pallas import tpu_sc as plsc

N = 10000
D = 128
F = 128
E = 320000

NC = 2
NS = 16
NW = NC * NS
L = 16

EW = E // NW
C = 80
NCHUNK = EW // C
GC = 25
NG = NCHUNK // GC

RPS = 624
TAIL = N - NS * RPS
ZR = RPS // 24

MM_BLK = 1000


def _mm_body(x_ref, f_ref, o_ref):
    o_ref[...] = jnp.dot(x_ref[...], f_ref[...],
                         preferred_element_type=jnp.float32)


def _matmul(x, filters):
    return pl.pallas_call(
        _mm_body,
        grid=(N // MM_BLK,),
        in_specs=[
            pl.BlockSpec((MM_BLK, D), lambda i: (i, 0)),
            pl.BlockSpec((D, F), lambda i: (0, 0)),
        ],
        out_specs=pl.BlockSpec((MM_BLK, F), lambda i: (i, 0)),
        out_shape=jax.ShapeDtypeStruct((N, F), jnp.float32),
    )(x, filters)


def _add_body(p_ref, o_ref):
    o_ref[...] = p_ref[0] + p_ref[1]


def _combine(partials):
    return pl.pallas_call(
        _add_body,
        grid=(N // MM_BLK,),
        in_specs=[pl.BlockSpec((NC, MM_BLK, F), lambda i: (0, i, 0))],
        out_specs=pl.BlockSpec((MM_BLK, F), lambda i: (i, 0)),
        out_shape=jax.ShapeDtypeStruct((N, F), jnp.float32),
    )(partials)


def _sc_body(xf_hbm, cols_hbm, rows_hbm, ev_hbm, out_hbm,
             cols_v, rows_v, ev_v, buf0, buf1, buf2, zbuf, acc,
             sem0, sem1, sem2, ssem0, ssem1, ssem2):
    c = lax.axis_index("c")
    s = lax.axis_index("s")
    wid = c * NS + s

    zeros16 = jnp.zeros((L,), jnp.float32)

    def zero_row(i, _):
        for j in range(F // L):
            zbuf[i, pl.ds(j * L, L)] = zeros16
        return 0

    lax.fori_loop(0, ZR, zero_row, 0)
    for k in range(RPS // ZR):
        pltpu.sync_copy(zbuf, acc.at[pl.ds(s * RPS + k * ZR, ZR)])

    @pl.when(s == NS - 1)
    def _zero_tail():
        pltpu.sync_copy(zbuf.at[pl.ds(0, TAIL)], acc.at[pl.ds(NS * RPS, TAIL)])

    plsc.subcore_barrier()

    def scale(buf, i):
        def sgroup(g, _):
            ev16 = ev_v[i, pl.ds(g * L, L)]
            for k in range(L):
                gb = jnp.broadcast_to(ev16[k], (L,))
                e = g * L + k
                for j in range(F // L):
                    buf[e, pl.ds(j * L, L)] = buf[e, pl.ds(j * L, L)] * gb
            return 0

        lax.fori_loop(0, C // L, sgroup, 0)

    def start_gather(i, buf, sem):
        pltpu.async_copy(xf_hbm.at[cols_v.at[i]], buf, sem)

    def wait_gather(i, buf, sem):
        pltpu.make_async_copy(xf_hbm.at[cols_v.at[i]], buf, sem).wait()

    def start_scatter(i, buf, sem):
        pltpu.async_copy(buf, acc.at[rows_v.at[i]], sem, add=True)

    def wait_scatter(i, buf, sem):
        pltpu.make_async_copy(buf, acc.at[rows_v.at[i]], sem).wait()

    def group(g, _):
        pltpu.sync_copy(cols_hbm.at[wid, g], cols_v)
        pltpu.sync_copy(rows_hbm.at[wid, g], rows_v)
        pltpu.sync_copy(ev_hbm.at[wid, g], ev_v)

        B = (buf0, buf1, buf2)
        GS = (sem0, sem1, sem2)
        SS = (ssem0, ssem1, ssem2)

        def step(j, first, last):
            b = j % 3
            wait_gather(j, B[b], GS[b])
            scale(B[b], j)
            start_scatter(j, B[b], SS[b])
            if not first:
                wait_scatter(j - 1, B[(b + 2) % 3], SS[(b + 2) % 3])
            if not last:
                start_gather(j + 2, B[(b + 2) % 3], GS[(b + 2) % 3])

        start_gather(0, buf0, sem0)
        start_gather(1, buf1, sem1)
        step(0, True, False)
        step(1, False, False)

        def triple(t, _):
            j = 3 * t + 2
            for d in range(3):
                b = (2 + d) % 3
                wait_gather(j + d, B[b], GS[b])
                scale(B[b], j + d)
                start_scatter(j + d, B[b], SS[b])
                wait_scatter(j + d - 1, B[(b + 2) % 3], SS[(b + 2) % 3])
                start_gather(j + d + 2, B[(b + 2) % 3], GS[(b + 2) % 3])
            return 0

        lax.fori_loop(0, (GC - 4) // 3, triple, 0)
        step(GC - 2, False, True)
        step(GC - 1, False, True)
        wait_scatter(GC - 1, B[(GC - 1) % 3], SS[(GC - 1) % 3])
        return 0

    lax.fori_loop(0, NG, group, 0)

    plsc.subcore_barrier()
    plsc.subcore_barrier()
    pltpu.sync_copy(acc.at[pl.ds(s * RPS, RPS)],
                    out_hbm.at[c, pl.ds(s * RPS, RPS)])

    @pl.when(s == NS - 1)
    def _write_tail():
        pltpu.sync_copy(acc.at[pl.ds(NS * RPS, TAIL)],
                        out_hbm.at[c, pl.ds(NS * RPS, TAIL)])


_sc_kernel = functools.partial(
    pl.kernel,
    out_type=jax.ShapeDtypeStruct((NC, N, F), jnp.float32),
    mesh=plsc.VectorSubcoreMesh(core_axis_name="c", subcore_axis_name="s",
                                num_cores=NC, num_subcores=NS),
    scratch_types=[
        pltpu.VMEM((GC, C), jnp.int32),
        pltpu.VMEM((GC, C), jnp.int32),
        pltpu.VMEM((GC, C), jnp.float32),
        pltpu.VMEM((C, F), jnp.float32),
        pltpu.VMEM((C, F), jnp.float32),
        pltpu.VMEM((C, F), jnp.float32),
        pltpu.VMEM((ZR, F), jnp.float32),
        pltpu.VMEM_SHARED((N, F), jnp.float32),
        pltpu.SemaphoreType.DMA,
        pltpu.SemaphoreType.DMA,
        pltpu.SemaphoreType.DMA,
        pltpu.SemaphoreType.DMA,
        pltpu.SemaphoreType.DMA,
        pltpu.SemaphoreType.DMA,
    ],
)(_sc_body)


@jax.jit
def kernel(x, edge_index, edge_values, filters):
    xf = _matmul(x, filters)
    rows4 = edge_index[0].reshape(NW, NG, GC, C)
    cols4 = edge_index[1].reshape(NW, NG, GC, C)
    ev4 = edge_values.reshape(NW, NG, GC, C)
    partials = _sc_kernel(xf, cols4, rows4, ev4)
    return _combine(partials)
